# baseline (speedup 1.0000x reference)
import jax
import jax.numpy as jnp
from jax import lax
from jax.experimental import pallas as pl
from jax.experimental.pallas import tpu as pltpu
from jax.experimental.pallas import tpu_sc as plsc

_LANES = 16
_CH = 32
_NBUF = 3
_DRAIN = 2


def _sc_dims():
    try:
        info = plsc.get_sparse_core_info()
        return info.num_cores, info.num_subcores
    except Exception:
        return 2, 16


def _make_gather(n_rows, batch, t, d):
    nc, ns = _sc_dims()
    nw = nc * ns
    bpw = batch // nw
    nch = bpw // _CH
    mesh = plsc.VectorSubcoreMesh(core_axis_name="c", subcore_axis_name="s")

    def body(table_hbm, cls_hbm, attr_hbm, na_hbm, out_hbm,
             cls_v, attr_v, idx_refs, na_v, bufs, gsems, wsems):
        wid = lax.axis_index("s") * nc + lax.axis_index("c")
        base = wid * bpw

        c0 = pltpu.async_copy(cls_hbm.at[pl.ds(base, bpw)], cls_v, gsems[0])
        c1 = pltpu.async_copy(attr_hbm.at[pl.ds(base, bpw)], attr_v, gsems[1])
        c2 = pltpu.async_copy(na_hbm, na_v, gsems[2])
        c0.wait()
        c1.wait()
        c2.wait()
        na = na_v[...]

        gathers = [None] * nch
        writes = [None] * nch
        for c in range(nch + _DRAIN):
            if c < nch:
                bb = c % _NBUF
                if c >= _NBUF:
                    writes[c - _NBUF].wait()
                for j in range(_CH // _LANES):
                    s = pl.ds(c * _CH + j * _LANES, _LANES)
                    sj = pl.ds(j * _LANES, _LANES)
                    idx_refs[c][sj] = cls_v[s] * na + attr_v[s]
                gathers[c] = pltpu.async_copy(
                    table_hbm.at[idx_refs[c]],
                    bufs.at[bb], gsems[bb])
            d_c = c - _DRAIN
            if d_c >= 0:
                bd = d_c % _NBUF
                gathers[d_c].wait()
                writes[d_c] = pltpu.async_copy(
                    bufs.at[bd],
                    out_hbm.at[pl.ds(base + d_c * _CH, _CH)],
                    wsems[bd])
        for d_c in range(max(0, nch - _NBUF), nch):
            writes[d_c].wait()

    return pl.kernel(
        body,
        out_type=jax.ShapeDtypeStruct((batch, t, d), jnp.float32),
        mesh=mesh,
        scratch_types=[
            pltpu.VMEM((bpw,), jnp.int32),
            pltpu.VMEM((bpw,), jnp.int32),
            [pltpu.VMEM((_CH,), jnp.int32)] * (bpw // _CH),
            pltpu.VMEM((_LANES,), jnp.int32),
            pltpu.VMEM((_NBUF, _CH, t, d), jnp.float32),
            [pltpu.SemaphoreType.DMA] * _NBUF,
            [pltpu.SemaphoreType.DMA] * _NBUF,
        ],
    )


def kernel(attribute_embeddings, class_idx, attr_idx, n_attrs):
    n, t, d = attribute_embeddings.shape
    cls = class_idx.astype(jnp.int32)
    att = attr_idx.astype(jnp.int32)
    na16 = jnp.broadcast_to(
        jnp.asarray(n_attrs, jnp.int32).reshape(()), (_LANES,))
    batch = cls.shape[0]
    return _make_gather(n, batch, t, d)(attribute_embeddings, cls, att, na16)


# speedup vs baseline: 1.0108x; 1.0108x over previous
import jax
import jax.numpy as jnp
from jax import lax
from jax.experimental import pallas as pl
from jax.experimental.pallas import tpu as pltpu
from jax.experimental.pallas import tpu_sc as plsc

_LANES = 16
_CH = 32
_NBUF = 3
_DRAIN = 2


def _sc_dims():
    try:
        info = plsc.get_sparse_core_info()
        return info.num_cores, info.num_subcores
    except Exception:
        return 2, 16


def _make_gather(n_rows, batch, t, d):
    nc, ns = _sc_dims()
    nw = nc * ns
    bpw = batch // nw
    nch = bpw // _CH
    mesh = plsc.VectorSubcoreMesh(core_axis_name="c", subcore_axis_name="s")

    def body(table_hbm, cls_hbm, attr_hbm, na_hbm, out_hbm,
             cls_v, attr_v, idx_refs, na_v, bufs, gsems, wsems):
        wid = lax.axis_index("s") * nc + lax.axis_index("c")
        base = wid * bpw

        c0 = pltpu.async_copy(cls_hbm.at[pl.ds(base, bpw)], cls_v, gsems[0])
        c1 = pltpu.async_copy(attr_hbm.at[pl.ds(base, bpw)], attr_v, gsems[1])
        c2 = pltpu.async_copy(na_hbm, na_v, gsems[2])
        c0.wait()
        c1.wait()
        c2.wait()
        na = na_v[...]
        for c in range(nch):
            for j in range(_CH // _LANES):
                s = pl.ds(c * _CH + j * _LANES, _LANES)
                sj = pl.ds(j * _LANES, _LANES)
                idx_refs[c][sj] = cls_v[s] * na + attr_v[s]

        gathers = [None] * nch
        writes = [None] * nch
        for c in range(nch + _DRAIN):
            if c < nch:
                bb = c % _NBUF
                if c >= _NBUF:
                    writes[c - _NBUF].wait()
                gathers[c] = pltpu.async_copy(
                    table_hbm.at[idx_refs[c]],
                    bufs.at[bb], gsems[bb])
            d_c = c - _DRAIN
            if d_c >= 0:
                bd = d_c % _NBUF
                gathers[d_c].wait()
                writes[d_c] = pltpu.async_copy(
                    bufs.at[bd],
                    out_hbm.at[pl.ds(base + d_c * _CH, _CH)],
                    wsems[bd])
        for d_c in range(max(0, nch - _NBUF), nch):
            writes[d_c].wait()

    return pl.kernel(
        body,
        out_type=jax.ShapeDtypeStruct((batch, t, d), jnp.float32),
        mesh=mesh,
        scratch_types=[
            pltpu.VMEM((bpw,), jnp.int32),
            pltpu.VMEM((bpw,), jnp.int32),
            [pltpu.VMEM((_CH,), jnp.int32)] * (bpw // _CH),
            pltpu.VMEM((_LANES,), jnp.int32),
            pltpu.VMEM((_NBUF, _CH, t, d), jnp.float32),
            [pltpu.SemaphoreType.DMA] * _NBUF,
            [pltpu.SemaphoreType.DMA] * _NBUF,
        ],
    )


def kernel(attribute_embeddings, class_idx, attr_idx, n_attrs):
    n, t, d = attribute_embeddings.shape
    cls = class_idx.astype(jnp.int32)
    att = attr_idx.astype(jnp.int32)
    na16 = jnp.broadcast_to(
        jnp.asarray(n_attrs, jnp.int32).reshape(()), (_LANES,))
    batch = cls.shape[0]
    return _make_gather(n, batch, t, d)(attribute_embeddings, cls, att, na16)
